# baseline (speedup 1.0000x reference)
import jax
import jax.numpy as jnp
from jax import lax
from jax.experimental import pallas as pl
from jax.experimental.pallas import tpu as pltpu
from jax.experimental.pallas import tpu_sc as plsc

N = 10000
E = 160000
D = 256
H = 128
NC = 2
NS = 16
NPAD = 10240
RPT = NPAD // NS
EPT = E // NS
CHUNK = 50
NCHUNKS = EPT // CHUNK
NBUF = 3
NRND = NCHUNKS // NBUF
TAIL = NCHUNKS - NRND * NBUF
UNPACK_OFFS = list(range(0, CHUNK - 15, 16)) + (
    [CHUNK - 16] if CHUNK % 16 else [])
CHUNK_D = 80
NCHUNKS_D = EPT // CHUNK_D
WBT = 10
WBR = N // WBT

_f32 = jnp.float32


def _sc_mesh():
    return plsc.VectorSubcoreMesh(
        core_axis_name="c", subcore_axis_name="s", num_cores=NC,
        num_subcores=NS)


def _make_seg_body(with_deg):
    def body(*refs):
        it = iter(refs)
        y0_hbm, y1_hbm, pidx_hbm, zrows_hbm = [next(it) for _ in range(4)]
        z1d_hbm = next(it) if with_deg else None
        agg0_hbm, agg1_hbm = next(it), next(it)
        deg_hbm = next(it) if with_deg else None
        pidx = [next(it) for _ in range(NBUF)]
        sidx = [next(it) for _ in range(NBUF)]
        didx = [next(it) for _ in range(NBUF)]
        rows = [next(it) for _ in range(NBUF)]
        ones_v = next(it) if with_deg else None
        acc_sh = next(it)
        dacc_sh = next(it) if with_deg else None
        isem = [next(it) for _ in range(NBUF)]
        gsem = [next(it) for _ in range(NBUF)]
        ssem = [next(it) for _ in range(NBUF)]
        dsem = [next(it) for _ in range(NBUF)] if with_deg else None
        zsem = next(it)

        c = lax.axis_index("c")
        s = lax.axis_index("s")

        @pl.when(s < WBT)
        def _():
            pltpu.async_copy(zrows_hbm, acc_sh.at[pl.ds(s * WBR, WBR)], zsem)

        if with_deg:
            @pl.when((c == 0) & (s == 0))
            def _():
                pltpu.async_copy(z1d_hbm, dacc_sh, zsem)
            for off in UNPACK_OFFS:
                ones_v[pl.ds(off, 16)] = jnp.ones((16,), _f32)

        def start_pidx(i, b):
            pltpu.async_copy(pidx_hbm.at[s * NCHUNKS + i], pidx[b], isem[b])

        def wait_pidx(b):
            pltpu.make_async_copy(pidx_hbm.at[0], pidx[b], isem[b]).wait()

        def unpack(b):
            for off in UNPACK_OFFS:
                v = pidx[b][0, pl.ds(off, 16)]
                sidx[b][pl.ds(off, 16)] = v & 0xFFFF
                didx[b][pl.ds(off, 16)] = lax.shift_right_logical(v, 16)

        def start_gather(b):
            @pl.when(c == 0)
            def _():
                pltpu.async_copy(y0_hbm.at[sidx[b]], rows[b], gsem[b])

            @pl.when(c == 1)
            def _():
                pltpu.async_copy(y1_hbm.at[sidx[b]], rows[b], gsem[b])

        def wait_gather(b):
            pltpu.make_async_copy(y0_hbm.at[sidx[b]], rows[b], gsem[b]).wait()

        def start_scatter(b):
            pltpu.async_copy(rows[b], acc_sh.at[didx[b]], ssem[b], add=True)
            if with_deg:
                @pl.when(c == 0)
                def _():
                    pltpu.async_copy(ones_v, dacc_sh.at[didx[b]], dsem[b],
                                     add=True)

        def wait_scatter(b):
            pltpu.make_async_copy(rows[b], acc_sh.at[didx[b]], ssem[b]).wait()
            if with_deg:
                @pl.when(c == 0)
                def _():
                    pltpu.make_async_copy(ones_v, dacc_sh.at[didx[b]],
                                          dsem[b]).wait()

        for b in range(NBUF):
            start_pidx(b, b)
        for b in range(NBUF):
            wait_pidx(b)
            unpack(b)
            start_pidx(NBUF + b, b)
            start_gather(b)

        @pl.when(s < WBT)
        def _():
            pltpu.make_async_copy(zrows_hbm, acc_sh.at[pl.ds(0, WBR)],
                                  zsem).wait()

        if with_deg:
            @pl.when((c == 0) & (s == 0))
            def _():
                pltpu.make_async_copy(z1d_hbm, dacc_sh, zsem).wait()

        plsc.subcore_barrier()

        def rnd(r, carry):
            base = r * NBUF
            for b in range(NBUF):
                wait_gather(b)
                start_scatter(b)
            for b in range(NBUF):
                nxt = base + NBUF + b

                @pl.when(nxt < NCHUNKS)
                def _():
                    wait_scatter(b)
                    wait_pidx(b)
                    unpack(b)

                    @pl.when(nxt + NBUF < NCHUNKS)
                    def _():
                        start_pidx(nxt + NBUF, b)

                    start_gather(b)
            return carry

        lax.fori_loop(0, NRND, rnd, 0)

        for b in range(TAIL):
            wait_gather(b)
            start_scatter(b)
        for b in range(TAIL, NBUF):
            wait_scatter(b)
        for b in range(TAIL):
            wait_scatter(b)
        plsc.subcore_barrier()

        @pl.when(s < WBT)
        def _():
            rws = pl.ds(s * WBR, WBR)

            @pl.when(c == 0)
            def _():
                pltpu.sync_copy(acc_sh.at[rws], agg0_hbm.at[rws])

            @pl.when(c == 1)
            def _():
                pltpu.sync_copy(acc_sh.at[rws], agg1_hbm.at[rws])

        if with_deg:
            @pl.when((c == 0) & (s == 0))
            def _():
                pltpu.sync_copy(dacc_sh, deg_hbm)

    return body


def _make_seg_call(with_deg):
    out = [jax.ShapeDtypeStruct((N, H), _f32),
           jax.ShapeDtypeStruct((N, H), _f32)]
    if with_deg:
        out = out + [jax.ShapeDtypeStruct((N,), _f32)]
    scratch = (
        [pltpu.VMEM((1, CHUNK), jnp.int32) for _ in range(NBUF)]
        + [pltpu.VMEM((CHUNK,), jnp.int32) for _ in range(2 * NBUF)]
        + [pltpu.VMEM((CHUNK, H), _f32) for _ in range(NBUF)]
        + ([pltpu.VMEM((CHUNK,), _f32)] if with_deg else [])
        + [pltpu.VMEM_SHARED((N, H), _f32)]
        + ([pltpu.VMEM_SHARED((N,), _f32)] if with_deg else [])
        + [pltpu.SemaphoreType.DMA
           for _ in range((4 if with_deg else 3) * NBUF + 1)]
    )
    return pl.kernel(
        _make_seg_body(with_deg),
        out_type=out,
        mesh=_sc_mesh(),
        scratch_types=scratch,
    )


_seg_call = _make_seg_call(False)
_segdeg_call = _make_seg_call(True)


RB = 1000
GRID = N // RB

_H_OUT = [jax.ShapeDtypeStruct((N, H), _f32),
          jax.ShapeDtypeStruct((N, H), _f32)]
_H_SPECS = [pl.BlockSpec((RB, H), lambda i: (i, 0)),
            pl.BlockSpec((RB, H), lambda i: (i, 0))]


def _comb_body(a0_ref, a1_ref, deg_ref, h0_ref, h1_ref, b_ref,
               wn_ref, wr_ref, o0_ref, o1_ref):
    inv = 1.0 / jnp.maximum(deg_ref[...], 1.0)
    mean = jnp.concatenate([a0_ref[...], a1_ref[...]], axis=1) * inv
    h = jnp.concatenate([h0_ref[...], h1_ref[...]], axis=1)
    o = (jnp.dot(mean, wn_ref[...], preferred_element_type=_f32)
         + b_ref[...]
         + jnp.dot(h, wr_ref[...], preferred_element_type=_f32))
    o = jnp.maximum(o, 0.0)
    o0_ref[...] = o[:, :H]
    o1_ref[...] = o[:, H:]


_comb = pl.pallas_call(
    _comb_body,
    grid=(GRID,),
    in_specs=_H_SPECS + _H_SPECS[:1] * 0 + [
        pl.BlockSpec((RB, 1), lambda i: (i, 0)),
    ] + _H_SPECS + [
        pl.BlockSpec((1, D), lambda i: (0, 0)),
        pl.BlockSpec((D, D), lambda i: (0, 0)),
        pl.BlockSpec((D, D), lambda i: (0, 0)),
    ],
    out_specs=_H_SPECS,
    out_shape=_H_OUT,
)

FC_PAD = 128


def _final_body(a0_ref, a1_ref, deg_ref, h0_ref, h1_ref, b_ref,
                wn_ref, wr_ref, wfc_ref, bfc_ref, out_ref):
    inv = 1.0 / jnp.maximum(deg_ref[...], 1.0)
    mean = jnp.concatenate([a0_ref[...], a1_ref[...]], axis=1) * inv
    h = jnp.concatenate([h0_ref[...], h1_ref[...]], axis=1)
    h4 = (jnp.dot(mean, wn_ref[...], preferred_element_type=_f32)
          + b_ref[...]
          + jnp.dot(h, wr_ref[...], preferred_element_type=_f32))
    out_ref[...] = jnp.dot(h4, wfc_ref[...],
                           preferred_element_type=_f32) + bfc_ref[...]


_final = pl.pallas_call(
    _final_body,
    grid=(GRID,),
    in_specs=_H_SPECS + [
        pl.BlockSpec((RB, 1), lambda i: (i, 0)),
    ] + _H_SPECS + [
        pl.BlockSpec((1, D), lambda i: (0, 0)),
        pl.BlockSpec((D, D), lambda i: (0, 0)),
        pl.BlockSpec((D, D), lambda i: (0, 0)),
        pl.BlockSpec((D, FC_PAD), lambda i: (0, 0)),
        pl.BlockSpec((1, FC_PAD), lambda i: (0, 0)),
    ],
    out_specs=pl.BlockSpec((RB, FC_PAD), lambda i: (i, 0)),
    out_shape=jax.ShapeDtypeStruct((N, FC_PAD), _f32),
)


def kernel(x, edge_index, W_l1a_n, W_l1a_r, b_l1a, W_l1b_n, W_l1b_r, b_l1b,
           W_l2a_n, W_l2a_r, b_l2a, W_l2b_n, W_l2b_r, b_l2b, W_fc, b_fc):
    src = edge_index[0]
    dst = edge_index[1]
    pidx = (src | (dst << 16)).reshape(NS * NCHUNKS, 1, CHUNK)
    zrows = jnp.zeros((WBR, H), _f32)
    z1d = jnp.zeros((N,), _f32)

    wfc_pad = jnp.zeros((D, FC_PAD), _f32).at[:, :2].set(W_fc)
    bfc_pad = jnp.zeros((1, FC_PAD), _f32).at[0, :2].set(b_fc)

    h0, h1 = x[:, :H], x[:, H:]
    a0, a1, deg = _segdeg_call(h0, h1, pidx, zrows, z1d)
    deg = deg.reshape(N, 1)
    h0, h1 = _comb(a0, a1, deg, h0, h1, b_l1a.reshape(1, D),
                   W_l1a_n, W_l1a_r)
    a0, a1 = _seg_call(h0, h1, pidx, zrows)
    h0, h1 = _comb(a0, a1, deg, h0, h1, b_l1b.reshape(1, D),
                   W_l1b_n, W_l1b_r)
    a0, a1 = _seg_call(h0, h1, pidx, zrows)
    h0, h1 = _comb(a0, a1, deg, h0, h1, b_l2a.reshape(1, D),
                   W_l2a_n, W_l2a_r)
    a0, a1 = _seg_call(h0, h1, pidx, zrows)
    out = _final(a0, a1, deg, h0, h1, b_l2b.reshape(1, D),
                 W_l2b_n, W_l2b_r, wfc_pad, bfc_pad)
    return out[:, :2]
